# Initial kernel scaffold; baseline (speedup 1.0000x reference)
import functools

import jax
import jax.numpy as jnp
from jax import lax
from jax.experimental import pallas as pl
from jax.experimental.pallas import tpu as pltpu
from jax.experimental.pallas import tpu_sc as plsc

_NC = 2
_NS = 16
_NW = _NC * _NS
_L = 16

_CHUNK = 80


def _precompute_tables(x, wa, wb, b1row):
    n = x.shape[0]
    blk = 1250
    d_in = x.shape[1]
    d_h = wa.shape[1]

    def body(x_ref, wa_ref, wb_ref, b1_ref, a_ref, b_ref):
        xv = x_ref[...]
        a_ref[...] = (
            jnp.dot(xv, wa_ref[...], preferred_element_type=jnp.float32)
            + b1_ref[...]
        )
        b_ref[...] = jnp.dot(xv, wb_ref[...], preferred_element_type=jnp.float32)

    return pl.pallas_call(
        body,
        grid=(n // blk,),
        in_specs=[
            pl.BlockSpec((blk, d_in), lambda i: (i, 0)),
            pl.BlockSpec((d_in, d_h), lambda i: (0, 0)),
            pl.BlockSpec((d_in, d_h), lambda i: (0, 0)),
            pl.BlockSpec((1, d_h), lambda i: (0, 0)),
        ],
        out_specs=[
            pl.BlockSpec((blk, d_h), lambda i: (i, 0)),
            pl.BlockSpec((blk, d_h), lambda i: (i, 0)),
        ],
        out_shape=[
            jax.ShapeDtypeStruct((n, d_h), jnp.float32),
            jax.ShapeDtypeStruct((n, d_h), jnp.float32),
        ],
    )(x, wa, wb, b1row)


def _edge_score_sc(tab_a, tab_b, row, col, params):
    n_edges = row.shape[0]
    d_h = tab_a.shape[1]
    per_w = n_edges // _NW
    n_chunks = per_w // _CHUNK
    n_groups = _CHUNK // _L
    mesh = plsc.VectorSubcoreMesh(core_axis_name="c", subcore_axis_name="s")

    @functools.partial(
        pl.kernel,
        out_type=jax.ShapeDtypeStruct((n_edges,), jnp.float32),
        mesh=mesh,
        scratch_types=[
            pltpu.VMEM((_CHUNK,), jnp.int32),
            pltpu.VMEM((_CHUNK,), jnp.int32),
            pltpu.VMEM((_CHUNK, d_h), jnp.float32),
            pltpu.VMEM((_CHUNK, d_h), jnp.float32),
            pltpu.VMEM((_CHUNK,), jnp.float32),
            pltpu.VMEM((80,), jnp.float32),
            pltpu.SemaphoreType.DMA,
            pltpu.SemaphoreType.DMA,
        ],
    )
    def k(a_hbm, b_hbm, row_hbm, col_hbm, par_hbm, out_hbm,
          idxa, idxb, bufa, bufb, outv, parv, sema, semb):
        wid = lax.axis_index("s") * _NC + lax.axis_index("c")
        wbase = wid * per_w
        pltpu.sync_copy(par_hbm, parv)
        b2v = plsc.load_gather(parv, [jnp.full((_L,), d_h, jnp.int32)])

        @pl.loop(0, n_chunks)
        def _chunk(i):
            base = wbase + i * _CHUNK
            pltpu.sync_copy(row_hbm.at[pl.ds(base, _CHUNK)], idxa)
            pltpu.sync_copy(col_hbm.at[pl.ds(base, _CHUNK)], idxb)
            cpa = pltpu.async_copy(a_hbm.at[idxa], bufa, sema)
            cpb = pltpu.async_copy(b_hbm.at[idxb], bufb, semb)
            cpa.wait()
            cpb.wait()
            for g in range(n_groups):
                eids = lax.iota(jnp.int32, _L) + (g * _L)

                def dbody(d, acc):
                    dd = jnp.zeros((_L,), jnp.int32) + d
                    av = plsc.load_gather(bufa, [eids, dd])
                    bv = plsc.load_gather(bufb, [eids, dd])
                    wv = plsc.load_gather(parv, [dd])
                    return acc + wv * jnp.maximum(av + bv, 0.0)

                z = lax.fori_loop(0, d_h, dbody, b2v)
                outv[pl.ds(g * _L, _L)] = 1.0 / (1.0 + jnp.exp(-z))
            pltpu.sync_copy(outv, out_hbm.at[pl.ds(base, _CHUNK)])

    return k(tab_a, tab_b, row, col, params)


def kernel(x, edge_index, W1, b1, W2, b2):
    d_in = x.shape[1]
    wa = W1[:d_in]
    wb = W1[d_in:]
    b1row = b1.reshape(1, -1)
    tab_a, tab_b = _precompute_tables(x, wa, wb, b1row)
    row = edge_index[0]
    col = edge_index[1]
    pad = jnp.zeros((80 - W2.shape[0] - 1,), jnp.float32)
    params = jnp.concatenate([W2[:, 0], b2, pad])
    return _edge_score_sc(tab_a, tab_b, row, col, params)


# speedup vs baseline: 1.8637x; 1.8637x over previous
import functools

import jax
import jax.numpy as jnp
from jax import lax
from jax.experimental import pallas as pl
from jax.experimental.pallas import tpu as pltpu
from jax.experimental.pallas import tpu_sc as plsc

_NC = 2
_NS = 16
_NW = _NC * _NS
_L = 16

_CHUNK = 80


def _precompute_tables(x, wa, wb, b1row):
    n = x.shape[0]
    blk = 1000
    d_in = x.shape[1]
    d_h = wa.shape[1]

    def body(x_ref, wa_ref, wb_ref, b1_ref, a_ref, b_ref):
        xv = x_ref[...]
        a_ref[...] = (
            jnp.dot(xv, wa_ref[...], preferred_element_type=jnp.float32)
            + b1_ref[...]
        )
        b_ref[...] = jnp.dot(xv, wb_ref[...], preferred_element_type=jnp.float32)

    return pl.pallas_call(
        body,
        grid=(n // blk,),
        in_specs=[
            pl.BlockSpec((blk, d_in), lambda i: (i, 0)),
            pl.BlockSpec((d_in, d_h), lambda i: (0, 0)),
            pl.BlockSpec((d_in, d_h), lambda i: (0, 0)),
            pl.BlockSpec((1, d_h), lambda i: (0, 0)),
        ],
        out_specs=[
            pl.BlockSpec((blk, d_h), lambda i: (i, 0)),
            pl.BlockSpec((blk, d_h), lambda i: (i, 0)),
        ],
        out_shape=[
            jax.ShapeDtypeStruct((n, d_h), jnp.float32),
            jax.ShapeDtypeStruct((n, d_h), jnp.float32),
        ],
    )(x, wa, wb, b1row)


def _edge_score_sc(tab_a, tab_b, row, col, params):
    n_edges = row.shape[0]
    d_h = tab_a.shape[1]
    per_w = n_edges // _NW
    n_chunks = per_w // _CHUNK
    n_groups = _CHUNK // _L
    mesh = plsc.VectorSubcoreMesh(core_axis_name="c", subcore_axis_name="s")

    @functools.partial(
        pl.kernel,
        out_type=jax.ShapeDtypeStruct((n_edges,), jnp.float32),
        mesh=mesh,
        compiler_params=pltpu.CompilerParams(needs_layout_passes=False, use_tc_tiling_on_sc=False),
        scratch_types=[
            pltpu.VMEM((_CHUNK,), jnp.int32),
            pltpu.VMEM((_CHUNK,), jnp.int32),
            pltpu.VMEM((_CHUNK, d_h), jnp.float32),
            pltpu.VMEM((_CHUNK, d_h), jnp.float32),
            pltpu.VMEM((_CHUNK,), jnp.float32),
            pltpu.VMEM((72, _L), jnp.float32),
            pltpu.SemaphoreType.DMA,
            pltpu.SemaphoreType.DMA,
        ],
    )
    def k(a_hbm, b_hbm, row_hbm, col_hbm, par_hbm, out_hbm,
          idxa, idxb, bufa, bufb, outv, parv, sema, semb):
        wid = lax.axis_index("s") * _NC + lax.axis_index("c")
        wbase = wid * per_w
        pltpu.sync_copy(par_hbm, parv)
        b2v = parv[d_h, :]

        @pl.loop(0, n_chunks)
        def _chunk(i):
            base = wbase + i * _CHUNK
            pltpu.sync_copy(row_hbm.at[pl.ds(base, _CHUNK)], idxa)
            pltpu.sync_copy(col_hbm.at[pl.ds(base, _CHUNK)], idxb)
            cpa = pltpu.async_copy(a_hbm.at[idxa], bufa, sema)
            cpb = pltpu.async_copy(b_hbm.at[idxb], bufb, semb)
            cpa.wait()
            cpb.wait()
            for g in range(n_groups):
                eids = lax.iota(jnp.int32, _L) + (g * _L)

                def dbody(d, acc):
                    dd = jnp.zeros((_L,), jnp.int32) + d
                    av = plsc.load_gather(bufa, [eids, dd])
                    bv = plsc.load_gather(bufb, [eids, dd])
                    wv = parv[d, :]
                    return acc + wv * jnp.maximum(av + bv, 0.0)

                z = lax.fori_loop(0, d_h, dbody, b2v)
                outv[pl.ds(g * _L, _L)] = 1.0 / (1.0 + jnp.exp(-z))
            pltpu.sync_copy(outv, out_hbm.at[pl.ds(base, _CHUNK)])

    return k(tab_a, tab_b, row, col, params)


def kernel(x, edge_index, W1, b1, W2, b2):
    d_in = x.shape[1]
    wa = W1[:d_in]
    wb = W1[d_in:]
    b1row = b1.reshape(1, -1)
    tab_a, tab_b = _precompute_tables(x, wa, wb, b1row)
    row = edge_index[0]
    col = edge_index[1]
    pad = jnp.zeros((72 - W2.shape[0] - 1,), jnp.float32)
    pvec = jnp.concatenate([W2[:, 0], b2, pad])
    params = jnp.broadcast_to(pvec[:, None], (72, _L)).astype(jnp.float32)
    return _edge_score_sc(tab_a, tab_b, row, col, params)


# speedup vs baseline: 2.5476x; 1.3669x over previous
import functools

import jax
import jax.numpy as jnp
from jax import lax
from jax.experimental import pallas as pl
from jax.experimental.pallas import tpu as pltpu
from jax.experimental.pallas import tpu_sc as plsc

_NC = 2
_NS = 16
_NW = _NC * _NS
_L = 16

_CHUNK = 400
_SUB = 80


def _precompute_tables(x, wa, wb, b1row):
    n = x.shape[0]
    blk = 1000
    d_in = x.shape[1]
    d_h = wa.shape[1]

    def body(x_ref, wa_ref, wb_ref, b1_ref, a_ref, b_ref):
        xv = x_ref[...]
        a_ref[...] = (
            jnp.dot(xv, wa_ref[...], preferred_element_type=jnp.float32)
            + b1_ref[...]
        )
        b_ref[...] = jnp.dot(xv, wb_ref[...], preferred_element_type=jnp.float32)

    return pl.pallas_call(
        body,
        grid=(n // blk,),
        in_specs=[
            pl.BlockSpec((blk, d_in), lambda i: (i, 0)),
            pl.BlockSpec((d_in, d_h), lambda i: (0, 0)),
            pl.BlockSpec((d_in, d_h), lambda i: (0, 0)),
            pl.BlockSpec((1, d_h), lambda i: (0, 0)),
        ],
        out_specs=[
            pl.BlockSpec((blk, d_h), lambda i: (i, 0)),
            pl.BlockSpec((blk, d_h), lambda i: (i, 0)),
        ],
        out_shape=[
            jax.ShapeDtypeStruct((n, d_h), jnp.float32),
            jax.ShapeDtypeStruct((n, d_h), jnp.float32),
        ],
    )(x, wa, wb, b1row)


def _edge_score_sc(tab_a, tab_b, row, col, params):
    n_edges = row.shape[0]
    d_h = tab_a.shape[1]
    per_w = n_edges // _NW
    n_chunks = per_w // _CHUNK
    n_groups = _CHUNK // _L
    n_sub = _CHUNK // _SUB
    mesh = plsc.VectorSubcoreMesh(core_axis_name="c", subcore_axis_name="s")

    @functools.partial(
        pl.kernel,
        out_type=jax.ShapeDtypeStruct((n_edges,), jnp.float32),
        mesh=mesh,
        compiler_params=pltpu.CompilerParams(
            needs_layout_passes=False, use_tc_tiling_on_sc=False
        ),
        scratch_types=[
            pltpu.VMEM((per_w,), jnp.int32),
            pltpu.VMEM((per_w,), jnp.int32),
            pltpu.VMEM((2 * _CHUNK, d_h), jnp.float32),
            pltpu.VMEM((2 * _CHUNK, d_h), jnp.float32),
            pltpu.VMEM((2 * _CHUNK,), jnp.float32),
            pltpu.VMEM((72, _L), jnp.float32),
            pltpu.SemaphoreType.DMA,
            pltpu.SemaphoreType.DMA,
            pltpu.SemaphoreType.DMA,
            pltpu.SemaphoreType.DMA,
        ],
    )
    def k(a_hbm, b_hbm, row_hbm, col_hbm, par_hbm, out_hbm,
          idxa, idxb, bufa, bufb, outv, parv, semg0, semg1, semo0, semo1):
        wid = lax.axis_index("s") * _NC + lax.axis_index("c")
        wbase = wid * per_w
        semg = (semg0, semg1)
        semo = (semo0, semo1)
        pltpu.sync_copy(par_hbm, parv)
        pltpu.sync_copy(row_hbm.at[pl.ds(wbase, per_w)], idxa)
        pltpu.sync_copy(col_hbm.at[pl.ds(wbase, per_w)], idxb)
        b2v = parv[d_h, :]

        def issue(ci, sl):
            for j in range(n_sub):
                off = ci * _CHUNK + j * _SUB
                dst = pl.ds(sl * _CHUNK + j * _SUB, _SUB)
                pltpu.async_copy(
                    a_hbm.at[idxa.at[pl.ds(off, _SUB)]],
                    bufa.at[dst, :], semg[sl])
                pltpu.async_copy(
                    b_hbm.at[idxb.at[pl.ds(off, _SUB)]],
                    bufb.at[dst, :], semg[sl])

        def wait_gathers(sl):
            sli = pl.ds(sl * _CHUNK, _CHUNK)
            pltpu.make_async_copy(
                a_hbm.at[idxa.at[pl.ds(0, _CHUNK)]], bufa.at[sli, :],
                semg[sl]).wait()
            pltpu.make_async_copy(
                b_hbm.at[idxb.at[pl.ds(0, _CHUNK)]], bufb.at[sli, :],
                semg[sl]).wait()

        def out_copy(ci, sl):
            pltpu.async_copy(
                outv.at[pl.ds(sl * _CHUNK, _CHUNK)],
                out_hbm.at[pl.ds(wbase + ci * _CHUNK, _CHUNK)], semo[sl])

        def wait_out(sl):
            pltpu.make_async_copy(
                outv.at[pl.ds(sl * _CHUNK, _CHUNK)],
                out_hbm.at[pl.ds(wbase, _CHUNK)], semo[sl]).wait()

        def compute(ci, sl):
            @pl.loop(0, n_groups)
            def _grp(g):
                eids = lax.iota(jnp.int32, _L) + (sl * _CHUNK + g * _L)
                acc = b2v
                for d in range(d_h):
                    dd = jnp.full((_L,), d, jnp.int32)
                    av = plsc.load_gather(bufa, [eids, dd])
                    bv = plsc.load_gather(bufb, [eids, dd])
                    wv = parv[d, :]
                    acc = acc + wv * jnp.maximum(av + bv, 0.0)
                outv[pl.ds(sl * _CHUNK + g * _L, _L)] = (
                    1.0 / (1.0 + jnp.exp(-acc)))

        def step(ci, sl, do_wait_out, do_issue):
            wait_gathers(sl)
            if do_wait_out:
                @pl.when(ci >= 2)
                def _():
                    wait_out(sl)
            compute(ci, sl)
            out_copy(ci, sl)
            if do_issue:
                @pl.when(ci + 2 < n_chunks)
                def _():
                    issue(ci + 2, sl)

        issue(0, 0)
        issue(1, 1)

        @pl.loop(0, n_chunks - 1, step=2)
        def _pair(i):
            step(i, 0, True, True)
            step(i + 1, 1, True, True)

        step(n_chunks - 1, 0, True, False)
        wait_out(0)
        wait_out(1)

    return k(tab_a, tab_b, row, col, params)


def kernel(x, edge_index, W1, b1, W2, b2):
    d_in = x.shape[1]
    wa = W1[:d_in]
    wb = W1[d_in:]
    b1row = b1.reshape(1, -1)
    tab_a, tab_b = _precompute_tables(x, wa, wb, b1row)
    row = edge_index[0]
    col = edge_index[1]
    pad = jnp.zeros((72 - W2.shape[0] - 1,), jnp.float32)
    pvec = jnp.concatenate([W2[:, 0], b2, pad])
    params = jnp.broadcast_to(pvec[:, None], (72, _L)).astype(jnp.float32)
    return _edge_score_sc(tab_a, tab_b, row, col, params)


# speedup vs baseline: 16.8180x; 6.6016x over previous
import functools

import jax
import jax.numpy as jnp
from jax import lax
from jax.experimental import pallas as pl
from jax.experimental.pallas import tpu as pltpu
from jax.experimental.pallas import tpu_sc as plsc

_NC = 2
_NS = 16
_NW = _NC * _NS
_L = 16

_CHUNK = 400
_SUB = 80


def _precompute_tables(x, W1, b1row):
    n = x.shape[0]
    blk = 1000
    d_in = x.shape[1]
    d_h = W1.shape[1]

    def body(x_ref, wa_ref, wb_ref, b1_ref, a_ref, b_ref):
        xv = x_ref[...].astype(jnp.bfloat16)
        a_ref[...] = (
            jnp.dot(xv, wa_ref[...].astype(jnp.bfloat16),
                    preferred_element_type=jnp.float32)
            + b1_ref[...]
        ).astype(jnp.bfloat16)
        b_ref[...] = jnp.dot(
            xv, wb_ref[...].astype(jnp.bfloat16),
            preferred_element_type=jnp.float32
        ).astype(jnp.bfloat16)

    return pl.pallas_call(
        body,
        grid=(n // blk,),
        in_specs=[
            pl.BlockSpec((blk, d_in), lambda i: (i, 0)),
            pl.BlockSpec((d_in, d_h), lambda i: (0, 0)),
            pl.BlockSpec((d_in, d_h), lambda i: (1, 0)),
            pl.BlockSpec((1, d_h), lambda i: (0, 0)),
        ],
        out_specs=[
            pl.BlockSpec((blk, d_h), lambda i: (i, 0)),
            pl.BlockSpec((blk, d_h), lambda i: (i, 0)),
        ],
        out_shape=[
            jax.ShapeDtypeStruct((n, d_h), jnp.bfloat16),
            jax.ShapeDtypeStruct((n, d_h), jnp.bfloat16),
        ],
    )(x, W1, W1, b1row)


def _edge_score_sc(tab_a, tab_b, edge_index, params, w2bf16):
    n_edges = edge_index.shape[1]
    d_h = tab_a.shape[1]
    per_w = n_edges // _NW
    n_chunks = per_w // _CHUNK
    n_groups = _CHUNK // _L
    n_sub = _CHUNK // _SUB
    mesh = plsc.VectorSubcoreMesh(core_axis_name="c", subcore_axis_name="s")

    @functools.partial(
        pl.kernel,
        out_type=jax.ShapeDtypeStruct((n_edges,), jnp.float32),
        mesh=mesh,
        compiler_params=pltpu.CompilerParams(
            needs_layout_passes=False, use_tc_tiling_on_sc=False
        ),
        scratch_types=[
            pltpu.VMEM((per_w,), jnp.int32),
            pltpu.VMEM((per_w,), jnp.int32),
            pltpu.VMEM((2 * _CHUNK, d_h), jnp.bfloat16),
            pltpu.VMEM((2 * _CHUNK, d_h), jnp.bfloat16),
            pltpu.VMEM((2 * _CHUNK,), jnp.float32),
            pltpu.VMEM((32,), jnp.float32),
            pltpu.VMEM((d_h,), jnp.bfloat16),
            pltpu.VMEM((2, 17 * _L), jnp.float32),
            pltpu.SemaphoreType.DMA,
            pltpu.SemaphoreType.DMA,
            pltpu.SemaphoreType.DMA,
            pltpu.SemaphoreType.DMA,
        ],
    )
    def k(a_hbm, b_hbm, ei_hbm, par_hbm, w2_hbm, out_hbm,
          idxa, idxb, bufa, bufb, outv, parv, w2bf, tbuf,
          semg0, semg1, semo0, semo1):
        wid = lax.axis_index("s") * _NC + lax.axis_index("c")
        wbase = wid * per_w
        semg = (semg0, semg1)
        semo = (semo0, semo1)
        pltpu.sync_copy(par_hbm, parv)
        pltpu.sync_copy(w2_hbm, w2bf)
        pltpu.sync_copy(ei_hbm.at[0, pl.ds(wbase, per_w)], idxa)
        pltpu.sync_copy(ei_hbm.at[1, pl.ds(wbase, per_w)], idxb)
        b2v = plsc.load_gather(parv, [jnp.zeros((_L,), jnp.int32)])

        def issue(ci, sl):
            for j in range(n_sub):
                off = ci * _CHUNK + j * _SUB
                dst = pl.ds(sl * _CHUNK + j * _SUB, _SUB)
                pltpu.async_copy(
                    a_hbm.at[idxa.at[pl.ds(off, _SUB)]],
                    bufa.at[dst, :], semg[sl])
                pltpu.async_copy(
                    b_hbm.at[idxb.at[pl.ds(off, _SUB)]],
                    bufb.at[dst, :], semg[sl])

        def wait_gathers(sl):
            sli = pl.ds(sl * _CHUNK, _CHUNK)
            pltpu.make_async_copy(
                a_hbm.at[idxa.at[pl.ds(0, _CHUNK)]], bufa.at[sli, :],
                semg[sl]).wait()
            pltpu.make_async_copy(
                b_hbm.at[idxb.at[pl.ds(0, _CHUNK)]], bufb.at[sli, :],
                semg[sl]).wait()

        def out_copy(ci, sl):
            pltpu.async_copy(
                outv.at[pl.ds(sl * _CHUNK, _CHUNK)],
                out_hbm.at[pl.ds(wbase + ci * _CHUNK, _CHUNK)], semo[sl])

        def wait_out(sl):
            pltpu.make_async_copy(
                outv.at[pl.ds(sl * _CHUNK, _CHUNK)],
                out_hbm.at[pl.ds(wbase, _CHUNK)], semo[sl]).wait()

        def compute(ci, sl):
            nc2 = d_h // (2 * _L)
            w2c = [w2bf[pl.ds(c2 * 2 * _L, 2 * _L)] for c2 in range(nc2)]

            def half_group(gbase, tb):
                accs = []
                for q in range(_L // 4):
                    sums = []
                    for e in range(4):
                        rowref_a = bufa.at[gbase + q * 4 + e]
                        rowref_b = bufb.at[gbase + q * 4 + e]
                        for c2 in range(nc2):
                            va = rowref_a[pl.ds(c2 * 2 * _L, 2 * _L)]
                            vb = rowref_b[pl.ds(c2 * 2 * _L, 2 * _L)]
                            sums.append(va + vb)
                    for e in range(4):
                        ts = [w2c[c2] * jnp.maximum(sums[e * nc2 + c2],
                                                    jnp.bfloat16(0.0))
                              for c2 in range(nc2)]
                        tsum = ts[0]
                        for t in ts[1:]:
                            tsum = tsum + t
                        u0, u1 = plsc.unpack(
                            tsum, format=plsc.PackFormat.INTERLEAVED)
                        accs.append(u0 + u1)
                for e in range(_L):
                    tb[pl.ds(e * 17, _L)] = accs[e]

            def half_sum(gbase, tb):
                lanes = lax.iota(jnp.int32, _L) * 17
                gs = [plsc.load_gather(tb, [lanes + d])
                      for d in range(_L)]
                while len(gs) > 1:
                    gs = [gs[i] + gs[i + 1] for i in range(0, len(gs), 2)]
                z = gs[0] + b2v
                outv[pl.ds(gbase, _L)] = 1.0 / (1.0 + jnp.exp(-z))

            @pl.loop(0, n_groups - 1, step=2)
            def _grp(g):
                gb0 = sl * _CHUNK + g * _L
                gb1 = gb0 + _L
                half_group(gb0, tbuf.at[0])
                half_group(gb1, tbuf.at[1])
                half_sum(gb0, tbuf.at[0])
                half_sum(gb1, tbuf.at[1])

            gtail = sl * _CHUNK + (n_groups - 1) * _L
            half_group(gtail, tbuf.at[0])
            half_sum(gtail, tbuf.at[0])

        def step(ci, sl, do_wait_out, do_issue):
            wait_gathers(sl)
            if do_wait_out:
                @pl.when(ci >= 2)
                def _():
                    wait_out(sl)
            compute(ci, sl)
            out_copy(ci, sl)
            if do_issue:
                @pl.when(ci + 2 < n_chunks)
                def _():
                    issue(ci + 2, sl)

        issue(0, 0)
        issue(1, 1)

        @pl.loop(0, n_chunks - 1, step=2)
        def _pair(i):
            step(i, 0, True, True)
            step(i + 1, 1, True, True)

        step(n_chunks - 1, 0, True, False)
        wait_out(0)
        wait_out(1)

    return k(tab_a, tab_b, edge_index, params, w2bf16)


def kernel(x, edge_index, W1, b1, W2, b2):
    b1row = b1.reshape(1, -1)
    tab_a, tab_b = _precompute_tables(x, W1, b1row)
    params = jnp.concatenate([b2, jnp.zeros((31,), jnp.float32)])
    w2bf16 = W2[:, 0].astype(jnp.bfloat16)
    return _edge_score_sc(tab_a, tab_b, edge_index, params, w2bf16)
